# Initial kernel scaffold; baseline (speedup 1.0000x reference)
import jax
import jax.numpy as jnp
from jax.experimental import pallas as pl


def kernel(edge_attr, edge_index, edge_type, pos, pos_w, k, params):
    raise NotImplementedError("write your pallas kernel here")


# speedup vs baseline: 1.9425x; 1.9425x over previous
import functools

import jax
import jax.numpy as jnp
from jax import lax
from jax.experimental import pallas as pl
from jax.experimental.pallas import tpu as pltpu
from jax.experimental.pallas import tpu_sc as plsc

HID = 64
NC, NS = 2, 16
NW = NC * NS
CH = 128
F32 = jnp.float32


def _silu(u):
    return u / (1.0 + jnp.exp(-u))


def _dot(a, b):
    return jnp.dot(a, b, preferred_element_type=F32)


def _edge_in_body(ea_ref, et_ref, wep_ref, bep_ref, lemb_ref, out_ref):
    ea = ea_ref[...]
    et = et_ref[...]
    onehot = (et == lax.broadcasted_iota(jnp.int32, (1, 15), 1)).astype(F32)
    out_ref[...] = (_dot(ea, wep_ref[...]) + _dot(onehot, lemb_ref[...])
                    + bep_ref[...])


def _xl_body(pos_ref, wg_ref, w_ref, b_ref, out_ref):
    out_ref[...] = (wg_ref[...] + _dot(pos_ref[...], w_ref[...])
                    + b_ref[...])


def _edge_layer_body(xs_ref, xd_ref, e_ref, we1_ref, be1_ref, we2_ref,
                     be2_ref, wm_ref, bm_ref, enew_ref, m_ref):
    xs, xd, e = xs_ref[...], xd_ref[...], e_ref[...]
    we1 = we1_ref[...]
    pre1 = (_dot(xs, we1[0:HID]) + _dot(xd, we1[HID:2 * HID])
            + _dot(e, we1[2 * HID:3 * HID]) + be1_ref[...])
    enew_ref[...] = e + _dot(_silu(pre1), we2_ref[...]) + be2_ref[...]
    wm = wm_ref[...]
    prem = (_dot(xs, wm[0:HID]) + _dot(xd, wm[HID:2 * HID])
            + _dot(e, wm[2 * HID:3 * HID]) + bm_ref[...])
    m_ref[...] = _silu(prem)


def _edge_layer3_body(xs_ref, xd_ref, e_ref, res_ref, we1_ref, be1_ref,
                      we2_ref, be2_ref, wo1_ref, bo1_ref, wo2_ref, bo2_ref,
                      pred_ref):
    xs, xd, e = xs_ref[...], xd_ref[...], e_ref[...]
    we1 = we1_ref[...]
    pre1 = (_dot(xs, we1[0:HID]) + _dot(xd, we1[HID:2 * HID])
            + _dot(e, we1[2 * HID:3 * HID]) + be1_ref[...])
    e3 = e + _dot(_silu(pre1), we2_ref[...]) + be2_ref[...]
    h = _silu(_dot(e3, wo1_ref[...]) + bo1_ref[...])
    pred_ref[...] = res_ref[...] + _dot(h, wo2_ref[...]) + bo2_ref[...]


def _node_body(x_ref, p0_ref, p1_ref, d0_ref, d1_ref, wn1_ref, bn1_ref,
               wn2_ref, bn2_ref, out_ref):
    x = x_ref[...]
    deg = jnp.maximum(d0_ref[...][:, 0:1] + d1_ref[...][:, 0:1], 1.0)
    agg = (p0_ref[...] + p1_ref[...]) / deg
    wn1 = wn1_ref[...]
    pre = _dot(x, wn1[0:HID]) + _dot(agg, wn1[HID:2 * HID]) + bn1_ref[...]
    out_ref[...] = x + _dot(_silu(pre), wn2_ref[...]) + bn2_ref[...]


def _sc_mesh():
    return plsc.VectorSubcoreMesh(core_axis_name="c", subcore_axis_name="s",
                                  num_cores=NC, num_subcores=NS)


@functools.lru_cache(maxsize=None)
def _make_gather1(V, M, D):
    n_chunks = M // CH
    per_w = -(-n_chunks // NW)

    @functools.partial(
        pl.kernel,
        out_type=jax.ShapeDtypeStruct((M, D), F32),
        mesh=_sc_mesh(),
        compiler_params=pltpu.CompilerParams(use_tc_tiling_on_sc=False),
        scratch_types=[
            pltpu.VMEM((CH,), jnp.int32),
            pltpu.VMEM((CH, D), F32),
            pltpu.SemaphoreType.DMA,
        ],
    )
    def gath(table_h, idx_h, out_h, idx_v, buf, sem):
        wid = lax.axis_index("s") * NC + lax.axis_index("c")

        def body(j, carry):
            c = j * NW + wid

            @pl.when(c < n_chunks)
            def _():
                base = pl.multiple_of(c * CH, CH)
                pltpu.sync_copy(idx_h.at[pl.ds(base, CH)], idx_v)
                pltpu.async_copy(table_h.at[idx_v], buf, sem).wait()
                pltpu.sync_copy(buf, out_h.at[pl.ds(base, CH)])
            return carry

        lax.fori_loop(0, per_w, body, 0)

    return gath


@functools.lru_cache(maxsize=None)
def _make_gather2(M, D):
    n_chunks = M // CH
    per_w = -(-n_chunks // NW)

    @functools.partial(
        pl.kernel,
        out_type=[jax.ShapeDtypeStruct((M, D), F32),
                  jax.ShapeDtypeStruct((M, D), F32)],
        mesh=_sc_mesh(),
        compiler_params=pltpu.CompilerParams(use_tc_tiling_on_sc=False),
        scratch_types=[
            pltpu.VMEM((CH,), jnp.int32),
            pltpu.VMEM((CH,), jnp.int32),
            pltpu.VMEM((CH, D), F32),
            pltpu.VMEM((CH, D), F32),
            pltpu.SemaphoreType.DMA,
            pltpu.SemaphoreType.DMA,
        ],
    )
    def gath(table_h, src_h, dst_h, xs_h, xd_h, idx_s, idx_d, buf_s, buf_d,
             sem_s, sem_d):
        wid = lax.axis_index("s") * NC + lax.axis_index("c")

        def body(j, carry):
            c = j * NW + wid

            @pl.when(c < n_chunks)
            def _():
                base = pl.multiple_of(c * CH, CH)
                pltpu.sync_copy(src_h.at[pl.ds(base, CH)], idx_s)
                pltpu.sync_copy(dst_h.at[pl.ds(base, CH)], idx_d)
                cp_s = pltpu.async_copy(table_h.at[idx_s], buf_s, sem_s)
                cp_d = pltpu.async_copy(table_h.at[idx_d], buf_d, sem_d)
                cp_s.wait()
                cp_d.wait()
                pltpu.sync_copy(buf_s, xs_h.at[pl.ds(base, CH)])
                pltpu.sync_copy(buf_d, xd_h.at[pl.ds(base, CH)])
            return carry

        lax.fori_loop(0, per_w, body, 0)

    return gath


@functools.lru_cache(maxsize=None)
def _make_scatter(M, NPAD, D):
    n_chunks = M // CH
    per_w = -(-n_chunks // NW)
    rows_pt = NPAD // NS

    @functools.partial(
        pl.kernel,
        out_type=[jax.ShapeDtypeStruct((NC, NPAD, D), F32),
                  jax.ShapeDtypeStruct((NC, NPAD, 16), F32)],
        mesh=_sc_mesh(),
        compiler_params=pltpu.CompilerParams(use_tc_tiling_on_sc=False),
        scratch_types=[
            pltpu.VMEM((CH,), jnp.int32),
            pltpu.VMEM((CH, D), F32),
            pltpu.VMEM((CH, 16), F32),
            pltpu.VMEM_SHARED((NPAD, D), F32),
            pltpu.VMEM_SHARED((NPAD, 16), F32),
            pltpu.SemaphoreType.DMA,
        ],
    )
    def scat(m_h, dst_h, z64_h, z16_h, out_h, outdeg_h, idx_v, buf, ones_v,
             acc_sh, dacc_sh, sem):
        cid = lax.axis_index("c")
        sid = lax.axis_index("s")
        wid = sid * NC + cid

        def fill_ones(r, carry):
            ones_v[r, :] = jnp.ones((16,), F32)
            return carry

        lax.fori_loop(0, CH, fill_ones, 0)

        @pl.when(sid == 0)
        def _():
            pltpu.sync_copy(z64_h, acc_sh)
            pltpu.sync_copy(z16_h, dacc_sh)

        plsc.subcore_barrier()

        def body(j, carry):
            c = j * NW + wid

            @pl.when(c < n_chunks)
            def _():
                base = pl.multiple_of(c * CH, CH)
                pltpu.sync_copy(dst_h.at[pl.ds(base, CH)], idx_v)
                pltpu.sync_copy(m_h.at[pl.ds(base, CH)], buf)
                pltpu.sync_copy(buf, acc_sh.at[idx_v], add=True)
                pltpu.sync_copy(ones_v, dacc_sh.at[idx_v], add=True)
            return carry

        lax.fori_loop(0, per_w, body, 0)
        plsc.subcore_barrier()

        r0 = sid * rows_pt
        pltpu.sync_copy(acc_sh.at[pl.ds(r0, rows_pt)],
                        out_h.at[cid, pl.ds(r0, rows_pt)])
        pltpu.sync_copy(dacc_sh.at[pl.ds(r0, rows_pt)],
                        outdeg_h.at[cid, pl.ds(r0, rows_pt)])

    return scat


def _pick_block(E, lo=1024, hi=2048):
    for b in range(hi, lo - 1, -8):
        if E % b == 0:
            return b
    return 2000


def kernel(edge_attr, edge_index, edge_type, pos, pos_w, k, params):
    p = params
    E = edge_attr.shape[0]
    N = pos.shape[0]
    NPAD = -(-N // CH) * CH
    BE = _pick_block(E)
    gridE = E // BE
    BN = NPAD // 16
    gridN = 16

    src = edge_index[0].astype(jnp.int32)
    dst = edge_index[1].astype(jnp.int32)
    et = edge_type.astype(jnp.int32).reshape(E, 1)
    res = edge_attr.reshape(E, 9, 5)[:, :, 4]
    pos_p = jnp.pad(pos, ((0, NPAD - N), (0, 0)))
    pos_w_p = jnp.pad(pos_w.astype(jnp.int32), (0, NPAD - N))
    w2sel = p['edge_out_w2'].reshape(HID, 9, 40)[:, :, k - 1]
    b2sel = p['edge_out_b2'].reshape(9, 40)[:, k - 1].reshape(1, 9)
    z64 = jnp.zeros((NPAD, HID), F32)
    z16 = jnp.zeros((NPAD, 16), F32)

    def row(b):
        return b.reshape(1, -1)

    full = lambda s: pl.BlockSpec(s, lambda i: tuple(0 for _ in s))
    eblk = lambda d: pl.BlockSpec((BE, d), lambda i: (i, 0))
    nblk = lambda d: pl.BlockSpec((BN, d), lambda i: (i, 0))

    wg = _make_gather1(p['wte'].shape[0], NPAD, HID)(p['wte'], pos_w_p)

    x = pl.pallas_call(
        _xl_body,
        grid=(gridN,),
        in_specs=[nblk(8), nblk(HID), full((8, HID)), full((1, HID))],
        out_specs=nblk(HID),
        out_shape=jax.ShapeDtypeStruct((NPAD, HID), F32),
    )(pos_p, wg, p['node_proj_w'], row(p['node_proj_b']))

    e = pl.pallas_call(
        _edge_in_body,
        grid=(gridE,),
        in_specs=[eblk(45), eblk(1), full((45, HID)), full((1, HID)),
                  full((15, HID))],
        out_specs=eblk(HID),
        out_shape=jax.ShapeDtypeStruct((E, HID), F32),
    )(edge_attr, et, p['edge_proj_w'], row(p['edge_proj_b']),
      p['layer_embed'])

    gather2 = _make_gather2(E, HID)
    scatter = _make_scatter(E, NPAD, HID)

    for li in range(2):
        lp = p['gnn'][li]
        xs, xd = gather2(x, src, dst)
        e, m = pl.pallas_call(
            _edge_layer_body,
            grid=(gridE,),
            in_specs=[eblk(HID), eblk(HID), eblk(HID),
                      full((3 * HID, HID)), full((1, HID)),
                      full((HID, HID)), full((1, HID)),
                      full((3 * HID, HID)), full((1, HID))],
            out_specs=[eblk(HID), eblk(HID)],
            out_shape=[jax.ShapeDtypeStruct((E, HID), F32),
                       jax.ShapeDtypeStruct((E, HID), F32)],
        )(xs, xd, e, lp['We1'], row(lp['be1']), lp['We2'], row(lp['be2']),
          lp['Wm'], row(lp['bm']))

        part, dpart = scatter(m, dst, z64, z16)

        x = pl.pallas_call(
            _node_body,
            grid=(gridN,),
            in_specs=[nblk(HID), nblk(HID), nblk(HID), nblk(16), nblk(16),
                      full((2 * HID, HID)), full((1, HID)),
                      full((HID, HID)), full((1, HID))],
            out_specs=nblk(HID),
            out_shape=jax.ShapeDtypeStruct((NPAD, HID), F32),
        )(x, part[0], part[1], dpart[0], dpart[1],
          lp['Wn1'], row(lp['bn1']), lp['Wn2'], row(lp['bn2']))

    lp = p['gnn'][2]
    xs, xd = gather2(x, src, dst)
    pred = pl.pallas_call(
        _edge_layer3_body,
        grid=(gridE,),
        in_specs=[eblk(HID), eblk(HID), eblk(HID), eblk(9),
                  full((3 * HID, HID)), full((1, HID)),
                  full((HID, HID)), full((1, HID)),
                  full((HID, HID)), full((1, HID)),
                  full((HID, 9)), full((1, 9))],
        out_specs=eblk(9),
        out_shape=jax.ShapeDtypeStruct((E, 9), F32),
    )(xs, xd, e, res, lp['We1'], row(lp['be1']), lp['We2'], row(lp['be2']),
      p['edge_out_w1'], row(p['edge_out_b1']), w2sel, b2sel)

    return pred
